# baseline (speedup 1.0000x reference)
import functools

import jax
import jax.numpy as jnp
from jax import lax
from jax.experimental import pallas as pl
from jax.experimental.pallas import tpu as pltpu

_K = 512
_LANES = 128

_PERM = (6, 7, 8, 3, 4, 5, 0, 1, 2, 9, 10, 11)


def _partner(arr, jp):
    n, L = arr.shape
    d = 1 << jp
    if d >= 8:
        a = arr.reshape(n // (2 * d), 2, d, L)
        return jnp.concatenate([a[:, 1:2], a[:, 0:1]], axis=1).reshape(n, L)
    i = lax.broadcasted_iota(jnp.int32, (n, 1), 0)
    bit_clear = ((i >> jp) & 1) == 0
    return jnp.where(bit_clear, jnp.roll(arr, -d, axis=0), jnp.roll(arr, d, axis=0))


def _cx(arr, jp, keep_max):
    p = _partner(arr, jp)
    return jnp.where(keep_max, jnp.maximum(arr, p), jnp.minimum(arr, p))


def _keep_max_stage(n, sp, jp):
    i = lax.broadcasted_iota(jnp.int32, (n, 1), 0)
    bitj = (i >> jp) & 1
    ascb = ((i >> sp) & 1) ^ 1
    return bitj == ascb


def _keep_max_desc(n, jp):
    i = lax.broadcasted_iota(jnp.int32, (n, 1), 0)
    return ((i >> jp) & 1) == 0


def _cx_desc(arr, jp):
    n, L = arr.shape
    d = 1 << jp
    if d >= 8:
        a = arr.reshape(n // (2 * d), 2, d, L)
        lo, hi = a[:, 0], a[:, 1]
        mx = jnp.maximum(lo, hi)
        mn = jnp.minimum(lo, hi)
        return jnp.concatenate([mx[:, None], mn[:, None]], axis=1).reshape(n, L)
    up = pltpu.roll(arr, n - d, 0)
    dn = pltpu.roll(arr, d, 0)
    return jnp.where(_keep_max_desc(n, jp), jnp.maximum(arr, up), jnp.minimum(arr, dn))


def _run_desc(a, q):
    G, m, L = a.shape
    half = m // 2
    lo, hi = a[:, :half], a[:, half:]
    mx = jnp.maximum(lo, hi)
    mn = jnp.minimum(lo, hi)
    if q == 1:
        return jnp.concatenate([mx, mn], axis=1)
    return jnp.concatenate([_run_desc(mx, q - 1), _run_desc(mn, q - 1)], axis=1)


def _merge_desc(arr, phys_bits):
    n, L = arr.shape
    i = 0
    while i < len(phys_bits):
        b = phys_bits[i]
        if (1 << b) >= 8:
            q = 1
            while (
                i + q < len(phys_bits)
                and phys_bits[i + q] == b - q
                and (1 << (b - q)) >= 8
            ):
                q += 1
            d0 = 1 << b
            a = arr.reshape(n // (2 * d0), 2 * d0, L)
            arr = _run_desc(a, q).reshape(n, L)
            i += q
        else:
            arr = _cx_desc(arr, b)
            i += 1
    return arr


def _sign_vectors(n):
    i = jnp.arange(n, dtype=jnp.int32)
    logk = _K.bit_length() - 1
    rows = [jnp.where(((i >> _PERM[1]) & 1) == 0, -1.0, 1.0)]
    for s in range(1, logk):
        b1, b2 = _PERM[s], _PERM[s + 1]
        rows.append(jnp.where((((i >> b1) ^ (i >> b2)) & 1) == 1, -1.0, 1.0))
    rows.append(jnp.where(((i >> _PERM[logk]) & 1) == 0, -1.0, 1.0))
    return jnp.stack(rows).astype(jnp.float32)


def _topk_block(arr, sgn):
    n, L = arr.shape
    logk = _K.bit_length() - 1

    arr = arr * sgn[0]
    for s in range(1, logk + 1):
        arr = _merge_desc(arr, [_PERM[j] for j in range(s - 1, -1, -1)])
        arr = arr * sgn[s]

    m = n // _K
    while m > 1:
        a = arr.reshape(m // 2, 2, _K, L)
        arr = jnp.maximum(a[:, 0], a[:, 1]).reshape((m // 2) * _K, L)
        m //= 2
        cur = m * _K
        if m > 1:
            arr = arr * sgn[logk, :cur]
        arr = _merge_desc(arr, [_PERM[j] for j in range(logk - 1, -1, -1)])
        if m > 1:
            arr = arr * sgn[logk, :cur]
    return arr


def _kernel_body(x_ref, s_ref, o_ref):
    o_ref[0] = _topk_block(x_ref[0], s_ref[...])


@jax.jit
def kernel(x):
    b, r, n = x.shape
    sgn = jnp.broadcast_to(_sign_vectors(n)[:, :, None], (10, n, _LANES))
    nc = 4
    bc = b // nc
    grid = (bc, r // _LANES)
    outs = []
    for c in range(nc):
        xt = jnp.swapaxes(x[c * bc:(c + 1) * bc], 1, 2)
        out_t = pl.pallas_call(
            _kernel_body,
            grid=grid,
            in_specs=[
                pl.BlockSpec((1, n, _LANES), lambda i, c: (i, 0, c)),
                pl.BlockSpec((10, n, _LANES), lambda i, c: (0, 0, 0)),
            ],
            out_specs=pl.BlockSpec((1, _K, _LANES), lambda i, c: (i, 0, c)),
            out_shape=jax.ShapeDtypeStruct((bc, _K, r), x.dtype),
        )(xt, sgn)
        out_t = out_t.reshape(bc, 8, 8, 8, r).transpose(0, 3, 2, 1, 4)
        outs.append(jnp.swapaxes(out_t.reshape(bc, _K, r), 1, 2))
    return jnp.concatenate(outs, axis=0)


# speedup vs baseline: 1.0939x; 1.0939x over previous
import functools

import jax
import jax.numpy as jnp
from jax import lax
from jax.experimental import pallas as pl
from jax.experimental.pallas import tpu as pltpu

_K = 512
_LANES = 128

_PERM = (6, 7, 8, 3, 4, 5, 0, 1, 2, 9, 10, 11)


def _partner(arr, jp):
    n, L = arr.shape
    d = 1 << jp
    if d >= 8:
        a = arr.reshape(n // (2 * d), 2, d, L)
        return jnp.concatenate([a[:, 1:2], a[:, 0:1]], axis=1).reshape(n, L)
    i = lax.broadcasted_iota(jnp.int32, (n, 1), 0)
    bit_clear = ((i >> jp) & 1) == 0
    return jnp.where(bit_clear, jnp.roll(arr, -d, axis=0), jnp.roll(arr, d, axis=0))


def _cx(arr, jp, keep_max):
    p = _partner(arr, jp)
    return jnp.where(keep_max, jnp.maximum(arr, p), jnp.minimum(arr, p))


def _keep_max_stage(n, sp, jp):
    i = lax.broadcasted_iota(jnp.int32, (n, 1), 0)
    bitj = (i >> jp) & 1
    ascb = ((i >> sp) & 1) ^ 1
    return bitj == ascb


def _keep_max_desc(n, jp):
    i = lax.broadcasted_iota(jnp.int32, (n, 1), 0)
    return ((i >> jp) & 1) == 0


def _cx_desc(arr, jp):
    n, L = arr.shape
    d = 1 << jp
    if d >= 8:
        a = arr.reshape(n // (2 * d), 2, d, L)
        lo, hi = a[:, 0], a[:, 1]
        mx = jnp.maximum(lo, hi)
        mn = jnp.minimum(lo, hi)
        return jnp.concatenate([mx[:, None], mn[:, None]], axis=1).reshape(n, L)
    up = pltpu.roll(arr, n - d, 0)
    dn = pltpu.roll(arr, d, 0)
    return jnp.where(_keep_max_desc(n, jp), jnp.maximum(arr, up), jnp.minimum(arr, dn))


def _run_desc(a, q, sg=None):
    G, m, L = a.shape
    half = m // 2
    lo, hi = a[:, :half], a[:, half:]
    mx = jnp.maximum(lo, hi)
    mn = jnp.minimum(lo, hi)
    sl = sh = None
    if sg is not None:
        sl, sh = sg[:, :half], sg[:, half:]
    if q == 1:
        if sg is not None:
            mx = mx * sl
            mn = mn * sh
        return jnp.concatenate([mx, mn], axis=1)
    return jnp.concatenate(
        [_run_desc(mx, q - 1, sl), _run_desc(mn, q - 1, sh)], axis=1
    )


def _merge_desc(arr, phys_bits, sg=None):
    n, L = arr.shape
    i = 0
    while i < len(phys_bits):
        b = phys_bits[i]
        if (1 << b) >= 8:
            q = 1
            while (
                i + q < len(phys_bits)
                and phys_bits[i + q] == b - q
                and (1 << (b - q)) >= 8
            ):
                q += 1
            d0 = 1 << b
            last = i + q == len(phys_bits)
            a = arr.reshape(n // (2 * d0), 2 * d0, L)
            sga = sg.reshape(n // (2 * d0), 2 * d0, L) if (sg is not None and last) else None
            arr = _run_desc(a, q, sga).reshape(n, L)
            i += q
        else:
            arr = _cx_desc(arr, b)
            if i + 1 == len(phys_bits) and sg is not None:
                arr = arr * sg
            i += 1
    return arr


def _sign_vectors(n):
    i = jnp.arange(n, dtype=jnp.int32)
    logk = _K.bit_length() - 1
    rows = [jnp.where(((i >> _PERM[1]) & 1) == 0, -1.0, 1.0)]
    for s in range(1, logk):
        b1, b2 = _PERM[s], _PERM[s + 1]
        rows.append(jnp.where((((i >> b1) ^ (i >> b2)) & 1) == 1, -1.0, 1.0))
    rows.append(jnp.where(((i >> _PERM[logk]) & 1) == 0, -1.0, 1.0))
    return jnp.stack(rows).astype(jnp.float32)


def _topk_block(arr, sgn):
    n, L = arr.shape
    logk = _K.bit_length() - 1

    arr = arr * sgn[0]
    for s in range(1, logk + 1):
        arr = _merge_desc(arr, [_PERM[j] for j in range(s - 1, -1, -1)], sgn[s])

    m = n // _K
    while m > 1:
        a = arr.reshape(m // 2, 2, _K, L)
        arr = jnp.maximum(a[:, 0], a[:, 1]).reshape((m // 2) * _K, L)
        m //= 2
        cur = m * _K
        if m > 1:
            arr = arr * sgn[logk, :cur]
        arr = _merge_desc(
            arr,
            [_PERM[j] for j in range(logk - 1, -1, -1)],
            sgn[logk, :cur] if m > 1 else None,
        )
    return arr


def _kernel_body(x_ref, s_ref, o_ref):
    o_ref[0] = _topk_block(x_ref[0], s_ref[...])


@jax.jit
def kernel(x):
    b, r, n = x.shape
    xt = jnp.swapaxes(x, 1, 2)
    sgn = jnp.broadcast_to(_sign_vectors(n)[:, :, None], (10, n, _LANES))
    grid = (b, r // _LANES)
    out_t = pl.pallas_call(
        _kernel_body,
        grid=grid,
        in_specs=[
            pl.BlockSpec((1, n, _LANES), lambda i, c: (i, 0, c)),
            pl.BlockSpec((10, n, _LANES), lambda i, c: (0, 0, 0)),
        ],
        out_specs=pl.BlockSpec((1, _K, _LANES), lambda i, c: (i, 0, c)),
        out_shape=jax.ShapeDtypeStruct((b, _K, r), x.dtype),
    )(xt, sgn)
    out_t = out_t.reshape(b, 8, 8, 8, r).transpose(0, 3, 2, 1, 4).reshape(b, _K, r)
    return jnp.swapaxes(out_t, 1, 2)


# speedup vs baseline: 1.1264x; 1.0297x over previous
import functools

import jax
import jax.numpy as jnp
from jax import lax
from jax.experimental import pallas as pl
from jax.experimental.pallas import tpu as pltpu

_K = 512
_LANES = 128

_PERM = (6, 7, 8, 3, 4, 5, 0, 1, 2, 9, 10, 11)


def _partner(arr, jp):
    n, L = arr.shape
    d = 1 << jp
    if d >= 8:
        a = arr.reshape(n // (2 * d), 2, d, L)
        return jnp.concatenate([a[:, 1:2], a[:, 0:1]], axis=1).reshape(n, L)
    i = lax.broadcasted_iota(jnp.int32, (n, 1), 0)
    bit_clear = ((i >> jp) & 1) == 0
    return jnp.where(bit_clear, jnp.roll(arr, -d, axis=0), jnp.roll(arr, d, axis=0))


def _cx(arr, jp, keep_max):
    p = _partner(arr, jp)
    return jnp.where(keep_max, jnp.maximum(arr, p), jnp.minimum(arr, p))


def _keep_max_stage(n, sp, jp):
    i = lax.broadcasted_iota(jnp.int32, (n, 1), 0)
    bitj = (i >> jp) & 1
    ascb = ((i >> sp) & 1) ^ 1
    return bitj == ascb


def _keep_max_desc(n, jp):
    i = lax.broadcasted_iota(jnp.int32, (n, 1), 0)
    return ((i >> jp) & 1) == 0


def _cx_desc(arr, jp):
    n, L = arr.shape
    d = 1 << jp
    if d >= 8:
        a = arr.reshape(n // (2 * d), 2, d, L)
        lo, hi = a[:, 0], a[:, 1]
        mx = jnp.maximum(lo, hi)
        mn = jnp.minimum(lo, hi)
        return jnp.concatenate([mx[:, None], mn[:, None]], axis=1).reshape(n, L)
    up = pltpu.roll(arr, n - d, 0)
    dn = pltpu.roll(arr, d, 0)
    return jnp.where(_keep_max_desc(n, jp), jnp.maximum(arr, up), jnp.minimum(arr, dn))


def _run_desc(a, q, sg=None):
    G, m, L = a.shape
    half = m // 2
    lo, hi = a[:, :half], a[:, half:]
    mx = jnp.maximum(lo, hi)
    mn = jnp.minimum(lo, hi)
    sl = sh = None
    if sg is not None:
        sl, sh = sg[:, :half], sg[:, half:]
    if q == 1:
        if sg is not None:
            mx = mx * sl
            mn = mn * sh
        return jnp.concatenate([mx, mn], axis=1)
    return jnp.concatenate(
        [_run_desc(mx, q - 1, sl), _run_desc(mn, q - 1, sh)], axis=1
    )


def _merge_desc(arr, phys_bits, sg=None):
    n, L = arr.shape
    i = 0
    while i < len(phys_bits):
        b = phys_bits[i]
        if (1 << b) >= 8:
            q = 1
            while (
                i + q < len(phys_bits)
                and phys_bits[i + q] == b - q
                and (1 << (b - q)) >= 8
            ):
                q += 1
            d0 = 1 << b
            last = i + q == len(phys_bits)
            a = arr.reshape(n // (2 * d0), 2 * d0, L)
            sga = sg.reshape(n // (2 * d0), 2 * d0, L) if (sg is not None and last) else None
            arr = _run_desc(a, q, sga).reshape(n, L)
            i += q
        else:
            arr = _cx_desc(arr, b)
            if i + 1 == len(phys_bits) and sg is not None:
                arr = arr * sg
            i += 1
    return arr


def _sign_vectors(n):
    i = jnp.arange(n, dtype=jnp.int32)
    logk = _K.bit_length() - 1
    rows = [jnp.where(((i >> _PERM[1]) & 1) == 0, -1.0, 1.0)]
    for s in range(1, logk):
        b1, b2 = _PERM[s], _PERM[s + 1]
        rows.append(jnp.where((((i >> b1) ^ (i >> b2)) & 1) == 1, -1.0, 1.0))
    rows.append(jnp.where(((i >> _PERM[logk]) & 1) == 0, -1.0, 1.0))
    return jnp.stack(rows).astype(jnp.float32)


def _topk_block(arr, sgn):
    n, L = arr.shape
    logk = _K.bit_length() - 1

    arr = arr * sgn[0]
    for s in range(1, logk + 1):
        arr = _merge_desc(arr, [_PERM[j] for j in range(s - 1, -1, -1)])
        arr = arr * sgn[s]

    m = n // _K
    while m > 1:
        a = arr.reshape(m // 2, 2, _K, L)
        arr = jnp.maximum(a[:, 0], a[:, 1]).reshape((m // 2) * _K, L)
        m //= 2
        cur = m * _K
        if m > 1:
            arr = arr * sgn[logk, :cur]
        arr = _merge_desc(arr, [_PERM[j] for j in range(logk - 1, -1, -1)])
        if m > 1:
            arr = arr * sgn[logk, :cur]
    return arr


def _kernel_body(x_ref, s_ref, o_ref):
    o_ref[0] = _topk_block(x_ref[0], s_ref[...])


@jax.jit
def kernel(x):
    b, r, n = x.shape
    xt = jnp.swapaxes(x, 1, 2)
    sgn = jnp.broadcast_to(_sign_vectors(n)[:, :, None], (10, n, _LANES))
    grid = (b, r // _LANES)
    out_t = pl.pallas_call(
        _kernel_body,
        grid=grid,
        in_specs=[
            pl.BlockSpec((1, n, _LANES), lambda i, c: (i, 0, c)),
            pl.BlockSpec((10, n, _LANES), lambda i, c: (0, 0, 0)),
        ],
        out_specs=pl.BlockSpec((1, _K, _LANES), lambda i, c: (i, 0, c)),
        out_shape=jax.ShapeDtypeStruct((b, _K, r), x.dtype),
    )(xt, sgn)
    out_t = out_t.reshape(b, 8, 8, 8, r).transpose(0, 3, 2, 1, 4).reshape(b, _K, r)
    return jnp.swapaxes(out_t, 1, 2)
